# baseline (speedup 1.0000x reference)
import functools

import jax
import jax.numpy as jnp
import numpy as np
from jax import lax
from jax.experimental import pallas as pl
from jax.experimental.pallas import tpu as pltpu

_BN_EPS = 1e-5
_INV_SQRT2 = np.float32(1.0 / np.sqrt(2.0))
_PAD = 128


def _gelu(y):
    return 0.5 * y * (1.0 + lax.erf(y * _INV_SQRT2))


def _taps_rmajor(yb, w):
    hw, c = yb.shape
    zpad = jnp.zeros((_PAD, c), yb.dtype)
    yp = jnp.concatenate([zpad, yb, zpad], axis=0)
    ri = lax.broadcasted_iota(jnp.int32, (hw + 2 * _PAD, c), 0) % w
    zero = jnp.zeros((), yb.dtype)
    yp_l = jnp.where(ri != (w - 1), yp, zero)
    yp_r = jnp.where(ri != 0, yp, zero)
    srcs = (yp_l, yp, yp_r)
    taps = []
    for dy in range(3):
        for dx in range(3):
            o = (dy - 1) * w + (dx - 1)
            taps.append(srcs[dx][_PAD + o:_PAD + o + hw, :])
    return taps


def _stats_rows(acc):
    s = jnp.sum(acc, axis=0, keepdims=True)
    ss = jnp.sum(acc * acc, axis=0, keepdims=True)
    return jnp.concatenate(
        [s, ss, jnp.zeros((6, acc.shape[1]), jnp.float32)], axis=0)


def _scale_shift(st_all, g, b, m):
    st = jnp.sum(st_all, axis=0)
    mean = st[0:1, :] / m
    var = jnp.maximum(st[1:2, :] / m - mean * mean, 0.0)
    inv = lax.rsqrt(var + _BN_EPS)
    scale = g * inv
    shift = b - mean * scale
    return scale, shift


def _conv1_kernel(x_ref, w_ref, o_ref, s_ref, *, w):
    xt = x_ref[0].astype(jnp.bfloat16)
    p = jnp.concatenate(_taps_rmajor(xt, w), axis=1)
    acc = jnp.dot(p, w_ref[...], preferred_element_type=jnp.float32)
    o_ref[0] = acc.astype(jnp.bfloat16)
    s_ref[0] = _stats_rows(acc)


def _conv2_kernel(h_ref, st_ref, g_ref, b_ref, w_ref, o_ref, s_ref, *, w, m):
    scale, shift = _scale_shift(st_ref[...], g_ref[...], b_ref[...], m)
    y = h_ref[0].astype(jnp.float32) * scale + shift
    yb = _gelu(y).astype(jnp.bfloat16)
    taps = _taps_rmajor(yb, w)
    cin = yb.shape[1]
    acc = None
    for dy in range(3):
        p = jnp.concatenate(taps[3 * dy:3 * dy + 3], axis=1)
        d = jnp.dot(p, w_ref[3 * dy * cin:(3 * dy + 3) * cin, :],
                    preferred_element_type=jnp.float32)
        acc = d if acc is None else acc + d
    o_ref[0] = acc.astype(jnp.bfloat16)
    s_ref[0] = _stats_rows(acc)


def _pool_kernel(h_ref, st_ref, g_ref, b_ref, o_ref, *, h, w, m):
    scale, shift = _scale_shift(st_ref[...], g_ref[...], b_ref[...], m)
    y = _gelu(h_ref[0].astype(jnp.float32) * scale + shift)
    c = y.shape[1]
    yb = y.astype(jnp.bfloat16)
    z = pltpu.bitcast(yb, jnp.int32)
    even = pltpu.bitcast(z << 16, jnp.float32)
    odd = pltpu.bitcast(z & jnp.int32(-65536), jnp.float32)
    wm = jnp.maximum(even, odd)
    wm = wm.reshape(h // 2, 2, (w // 2), c)
    hm = jnp.maximum(wm[:, 0], wm[:, 1])
    o_ref[0] = hm.reshape((h // 2) * (w // 2), c)


def kernel(x, w1, b1, g1, be1, w2, b2, g2, be2):
    n, cin, h, w = x.shape
    cout = g1.shape[0]
    hw = h * w
    m = float(n * hw)

    x2 = jnp.transpose(x.reshape(n, cin, hw), (0, 2, 1))
    w1m = jnp.transpose(w1, (2, 3, 1, 0)).reshape(9 * cin, cout).astype(jnp.bfloat16)
    w2m = jnp.transpose(w2, (2, 3, 1, 0)).reshape(9 * cout, cout).astype(jnp.bfloat16)
    g1c, be1c = g1.reshape(1, cout), be1.reshape(1, cout)
    g2c, be2c = g2.reshape(1, cout), be2.reshape(1, cout)

    cparams = pltpu.CompilerParams(
        dimension_semantics=("parallel",),
        vmem_limit_bytes=100 * 1024 * 1024,
    )

    conv1, st1 = pl.pallas_call(
        functools.partial(_conv1_kernel, w=w),
        grid=(n,),
        in_specs=[pl.BlockSpec((1, hw, cin), lambda i: (i, 0, 0)),
                  pl.BlockSpec((9 * cin, cout), lambda i: (0, 0))],
        out_specs=[pl.BlockSpec((1, hw, cout), lambda i: (i, 0, 0)),
                   pl.BlockSpec((1, 8, cout), lambda i: (i, 0, 0))],
        out_shape=[jax.ShapeDtypeStruct((n, hw, cout), jnp.bfloat16),
                   jax.ShapeDtypeStruct((n, 8, cout), jnp.float32)],
        compiler_params=cparams,
    )(x2, w1m)

    conv2, st2 = pl.pallas_call(
        functools.partial(_conv2_kernel, w=w, m=m),
        grid=(n,),
        in_specs=[pl.BlockSpec((1, hw, cout), lambda i: (i, 0, 0)),
                  pl.BlockSpec((n, 8, cout), lambda i: (0, 0, 0)),
                  pl.BlockSpec((1, cout), lambda i: (0, 0)),
                  pl.BlockSpec((1, cout), lambda i: (0, 0)),
                  pl.BlockSpec((9 * cout, cout), lambda i: (0, 0))],
        out_specs=[pl.BlockSpec((1, hw, cout), lambda i: (i, 0, 0)),
                   pl.BlockSpec((1, 8, cout), lambda i: (i, 0, 0))],
        out_shape=[jax.ShapeDtypeStruct((n, hw, cout), jnp.bfloat16),
                   jax.ShapeDtypeStruct((n, 8, cout), jnp.float32)],
        compiler_params=cparams,
    )(conv1, st1, g1c, be1c, w2m)

    out = pl.pallas_call(
        functools.partial(_pool_kernel, h=h, w=w, m=m),
        grid=(n,),
        in_specs=[pl.BlockSpec((1, hw, cout), lambda i: (i, 0, 0)),
                  pl.BlockSpec((n, 8, cout), lambda i: (0, 0, 0)),
                  pl.BlockSpec((1, cout), lambda i: (0, 0)),
                  pl.BlockSpec((1, cout), lambda i: (0, 0))],
        out_specs=pl.BlockSpec((1, hw // 4, cout), lambda i: (i, 0, 0)),
        out_shape=jax.ShapeDtypeStruct((n, hw // 4, cout), jnp.float32),
        compiler_params=cparams,
    )(conv2, st2, g2c, be2c)

    return jnp.transpose(out.reshape(n, h // 2, w // 2, cout), (0, 3, 1, 2))
